# baseline (speedup 1.0000x reference)
import jax
import jax.numpy as jnp
from jax.experimental import pallas as pl
from jax.experimental.pallas import tpu as pltpu

_KNN = 14
_BQ = 128
_L = 128


def _nld_body(q_ref, k_ref, qsq_ref, ksq_ref, o_ref, d_ref):
    BQ = q_ref.shape[0]
    K = k_ref.shape[0]
    C = K // _L
    q = q_ref[...]
    inf = jnp.float32(jnp.inf)

    _MC = 1024
    qsq = qsq_ref[...]
    m1 = m2 = m3 = m4 = jnp.full((BQ, _L), inf, jnp.float32)
    for ci in range(K // _MC):
        ks_c = k_ref[ci * _MC:(ci + 1) * _MC, :]
        qk_c = jax.lax.dot_general(
            q, ks_c, (((1,), (1,)), ((), ())),
            preferred_element_type=jnp.float32)
        dc = qsq + ksq_ref[:, ci * _MC:(ci + 1) * _MC] + qk_c
        d_ref[:, ci * _MC:(ci + 1) * _MC] = dc
        for j in range(_MC // _L):
            v = dc[:, j * _L:(j + 1) * _L]
            s = jnp.maximum(m1, v)
            m1 = jnp.minimum(m1, v)
            t = jnp.maximum(m2, s)
            m2 = jnp.minimum(m2, s)
            u = jnp.maximum(m3, t)
            m3 = jnp.minimum(m3, t)
            m4 = jnp.minimum(m4, u)
    cand = jnp.concatenate([m1, m2, m3, m4], axis=1)

    v1 = None
    t_cand = None
    cur = cand
    for i in range(_KNN):
        m = jnp.min(cur, axis=1, keepdims=True)
        if i == 0:
            v1 = m
        t_cand = m
        cur = jnp.where(cur == m, inf, cur)

    maybe_bad = jnp.any(m4 < t_cand)

    def full_extract(_):
        def fe_body(i, carry):
            tt, curf = carry
            mf = jnp.min(curf, axis=1, keepdims=True)
            return mf, jnp.where(curf == mf, inf, curf)
        tt, _ = jax.lax.fori_loop(0, _KNN, fe_body, (t_cand, d_ref[...]))
        return tt

    def confirm_and_fix(_):
        d2 = d_ref[...]
        cnt = jnp.sum(jnp.where(d2 < t_cand, 1.0, 0.0), axis=1, keepdims=True)
        bad = jnp.any(cnt > 13.5)
        return jax.lax.cond(bad, full_extract, lambda _: t_cand, 0)

    t_fin = jax.lax.cond(maybe_bad, confirm_and_fix, lambda _: t_cand, 0)

    denom = jnp.zeros((BQ, 1), jnp.float32)
    num = jnp.zeros((BQ, q.shape[1]), jnp.float32)
    for ci in range(K // _MC):
        d2c = d_ref[:, ci * _MC:(ci + 1) * _MC]
        wc = jnp.where(d2c <= t_fin, jnp.exp(v1 - d2c), 0.0)
        denom = denom + jnp.sum(wc, axis=1, keepdims=True)
        num = num + jax.lax.dot_general(
            wc, k_ref[ci * _MC:(ci + 1) * _MC, :], (((1,), (0,)), ((), ())),
            preferred_element_type=jnp.float32)
    o_ref[...] = num / denom


def kernel(queries, keys, k):
    Q, d = queries.shape
    K = keys.shape[0]
    q_sq = jnp.sum(queries * queries, axis=1, keepdims=True)
    k_sq = jnp.sum(keys * keys, axis=1)[None, :]
    q_n2 = queries * -2.0
    out = pl.pallas_call(
        _nld_body,
        grid=(Q // _BQ,),
        in_specs=[
            pl.BlockSpec((_BQ, d), lambda i: (i, 0)),
            pl.BlockSpec((K, d), lambda i: (0, 0)),
            pl.BlockSpec((_BQ, 1), lambda i: (i, 0)),
            pl.BlockSpec((1, K), lambda i: (0, 0)),
        ],
        out_specs=pl.BlockSpec((_BQ, d), lambda i: (i, 0)),
        out_shape=jax.ShapeDtypeStruct((Q, d), jnp.float32),
        scratch_shapes=[
            pltpu.VMEM((_BQ, K), jnp.float32),
        ],
    )(q_n2, keys, q_sq, k_sq)
    return out


# speedup vs baseline: 1.0164x; 1.0164x over previous
import jax
import jax.numpy as jnp
from jax.experimental import pallas as pl
from jax.experimental.pallas import tpu as pltpu

_KNN = 14
_BQ = 128
_L = 128


def _nld_body(q_ref, k_ref, qsq_ref, ksq_ref, o_ref, d_ref):
    BQ = q_ref.shape[0]
    K = k_ref.shape[0]
    C = K // _L
    q = q_ref[...]
    inf = jnp.float32(jnp.inf)

    _MC = 2048
    qsq = qsq_ref[...]
    m1 = m2 = m3 = m4 = jnp.full((BQ, _L), inf, jnp.float32)
    for ci in range(K // _MC):
        ks_c = k_ref[ci * _MC:(ci + 1) * _MC, :]
        qk_c = jax.lax.dot_general(
            q, ks_c, (((1,), (1,)), ((), ())),
            preferred_element_type=jnp.float32)
        dc = qsq + ksq_ref[:, ci * _MC:(ci + 1) * _MC] + qk_c
        d_ref[:, ci * _MC:(ci + 1) * _MC] = dc
        for j in range(_MC // _L):
            v = dc[:, j * _L:(j + 1) * _L]
            s = jnp.maximum(m1, v)
            m1 = jnp.minimum(m1, v)
            t = jnp.maximum(m2, s)
            m2 = jnp.minimum(m2, s)
            u = jnp.maximum(m3, t)
            m3 = jnp.minimum(m3, t)
            m4 = jnp.minimum(m4, u)
    cand = jnp.concatenate([m1, m2, m3, m4], axis=1)

    v1 = None
    t_cand = None
    cur = cand
    for i in range(_KNN):
        m = jnp.min(cur, axis=1, keepdims=True)
        if i == 0:
            v1 = m
        t_cand = m
        cur = jnp.where(cur == m, inf, cur)

    maybe_bad = jnp.any(m4 < t_cand)

    def full_extract(_):
        def fe_body(i, carry):
            tt, curf = carry
            mf = jnp.min(curf, axis=1, keepdims=True)
            return mf, jnp.where(curf == mf, inf, curf)
        tt, _ = jax.lax.fori_loop(0, _KNN, fe_body, (t_cand, d_ref[...]))
        return tt

    def confirm_and_fix(_):
        d2 = d_ref[...]
        cnt = jnp.sum(jnp.where(d2 < t_cand, 1.0, 0.0), axis=1, keepdims=True)
        bad = jnp.any(cnt > 13.5)
        return jax.lax.cond(bad, full_extract, lambda _: t_cand, 0)

    t_fin = jax.lax.cond(maybe_bad, confirm_and_fix, lambda _: t_cand, 0)

    denom = jnp.zeros((BQ, 1), jnp.float32)
    num = jnp.zeros((BQ, q.shape[1]), jnp.float32)
    for ci in range(K // _MC):
        d2c = d_ref[:, ci * _MC:(ci + 1) * _MC]
        wc = jnp.where(d2c <= t_fin, jnp.exp(v1 - d2c), 0.0)
        denom = denom + jnp.sum(wc, axis=1, keepdims=True)
        num = num + jax.lax.dot_general(
            wc, k_ref[ci * _MC:(ci + 1) * _MC, :], (((1,), (0,)), ((), ())),
            preferred_element_type=jnp.float32)
    o_ref[...] = num / denom


def kernel(queries, keys, k):
    Q, d = queries.shape
    K = keys.shape[0]
    q_sq = jnp.sum(queries * queries, axis=1, keepdims=True)
    k_sq = jnp.sum(keys * keys, axis=1)[None, :]
    q_n2 = queries * -2.0
    out = pl.pallas_call(
        _nld_body,
        grid=(Q // _BQ,),
        in_specs=[
            pl.BlockSpec((_BQ, d), lambda i: (i, 0)),
            pl.BlockSpec((K, d), lambda i: (0, 0)),
            pl.BlockSpec((_BQ, 1), lambda i: (i, 0)),
            pl.BlockSpec((1, K), lambda i: (0, 0)),
        ],
        out_specs=pl.BlockSpec((_BQ, d), lambda i: (i, 0)),
        out_shape=jax.ShapeDtypeStruct((Q, d), jnp.float32),
        scratch_shapes=[
            pltpu.VMEM((_BQ, K), jnp.float32),
        ],
    )(q_n2, keys, q_sq, k_sq)
    return out
